# baseline (speedup 1.0000x reference)
import jax
import jax.numpy as jnp
from jax import lax
from jax.experimental import pallas as pl
from jax.experimental.pallas import tpu as pltpu
from jax.experimental.pallas import tpu_sc as plsc

N = 10000
E = 320000
H = 128
NRBF = 32
L = 2
G = 64
CUT = 5.0
NT = 11

NC = 2
NS = 16
NW = NC * NS
EPT = E // NW
CB = 40
NCHUNK = EPT // CB
NPAD = 10240
NPT = NPAD // NS
WB = 40
NB = 2000
EB = 2560

_MESH = plsc.VectorSubcoreMesh(core_axis_name="c", subcore_axis_name="s")


def _geom_body(px_hbm, py_hbm, pz_hbm, row_hbm, col_hbm, d_hbm,
               px_v, py_v, pz_v, row_v, col_v, d_v):
  cid = lax.axis_index("c")
  sid = lax.axis_index("s")
  wid = sid * NC + cid
  base = wid * EPT
  pltpu.sync_copy(px_hbm, px_v)
  pltpu.sync_copy(py_hbm, py_v)
  pltpu.sync_copy(pz_hbm, pz_v)
  pltpu.sync_copy(row_hbm.at[pl.ds(base, EPT)], row_v)
  pltpu.sync_copy(col_hbm.at[pl.ds(base, EPT)], col_v)

  def body(i, carry):
    off = i * 16
    ir = row_v[pl.ds(off, 16)]
    ic = col_v[pl.ds(off, 16)]
    dx = plsc.load_gather(px_v, [ic]) - plsc.load_gather(px_v, [ir])
    dy = plsc.load_gather(py_v, [ic]) - plsc.load_gather(py_v, [ir])
    dz = plsc.load_gather(pz_v, [ic]) - plsc.load_gather(pz_v, [ir])
    s = dx * dx + dy * dy + dz * dz
    bits = plsc.bitcast(s, jnp.int32)
    y = plsc.bitcast(0x5F3759DF - (bits >> 1), jnp.float32)
    for _ in range(3):
      y = y * (1.5 - 0.5 * s * y * y)
    d_v[pl.ds(off, 16)] = jnp.maximum(s * y, 1e-8)
    return carry

  lax.fori_loop(0, EPT // 16, body, 0)
  pltpu.sync_copy(d_v, d_hbm.at[pl.ds(base, EPT)])


_geom = pl.kernel(
    _geom_body,
    out_type=jax.ShapeDtypeStruct((E,), jnp.float32),
    mesh=_MESH,
    compiler_params=pltpu.CompilerParams(needs_layout_passes=False),
    scratch_types=[
        pltpu.VMEM((N,), jnp.float32),
        pltpu.VMEM((N,), jnp.float32),
        pltpu.VMEM((N,), jnp.float32),
        pltpu.VMEM((EPT,), jnp.int32),
        pltpu.VMEM((EPT,), jnp.int32),
        pltpu.VMEM((EPT,), jnp.float32),
    ],
)


def _edge_body(h0_hbm, g1_hbm, ew_hbm, row_hbm, col_hbm, out_hbm,
               rowc0, colc0, ew0, h0b0, g1b0,
               rowc1, colc1, ew1, h0b1, g1b1,
               m_v, agg_sh, ewsem0, gsem0, hsem0, ewsem1, gsem1, hsem1):
  rowc = (rowc0, rowc1)
  colc = (colc0, colc1)
  ewv = (ew0, ew1)
  h0v = (h0b0, h0b1)
  g1v = (g1b0, g1b1)
  ewsem = (ewsem0, ewsem1)
  gsem = (gsem0, gsem1)
  hsem = (hsem0, hsem1)
  cid = lax.axis_index("c")
  sid = lax.axis_index("s")
  wid = sid * NC + cid

  def zb(i, carry):
    for j in range(H // 16):
      h0b0[i, pl.ds(16 * j, 16)] = jnp.zeros((16,), jnp.float32)
    return carry

  lax.fori_loop(0, WB, zb, 0)
  for k in range(NPT // WB):
    pltpu.sync_copy(h0b0, agg_sh.at[pl.ds(sid * NPT + k * WB, WB)])
  plsc.subcore_barrier()

  def fire(ci, b):
    e0 = wid * EPT + ci * CB
    pltpu.sync_copy(row_hbm.at[pl.ds(e0, CB)], rowc[b])
    pltpu.sync_copy(col_hbm.at[pl.ds(e0, CB)], colc[b])
    pltpu.async_copy(ew_hbm.at[pl.ds(e0, CB)], ewv[b], ewsem[b])
    pltpu.async_copy(g1_hbm.at[rowc[b]], g1v[b], gsem[b])
    pltpu.async_copy(h0_hbm.at[colc[b]], h0v[b], hsem[b])

  fire(0, 0)

  def pair(k, carry):
    for b in range(2):
      ci = 2 * k + b
      nb = 1 - b

      @pl.when(ci + 1 < NCHUNK)
      def _():
        fire(ci + 1, nb)

      pltpu.make_async_copy(ew_hbm.at[pl.ds(0, CB)], ewv[b], ewsem[b]).wait()
      pltpu.make_async_copy(g1_hbm.at[rowc[b]], g1v[b], gsem[b]).wait()
      pltpu.make_async_copy(h0_hbm.at[colc[b]], h0v[b], hsem[b]).wait()

      @plsc.parallel_loop(0, CB, 1, unroll=2)
      def _(e):
        for j in range(H // 16):
          h = h0v[b][e, pl.ds(16 * j, 16)]
          f = ewv[b][e, pl.ds(16 * j, 16)]
          gn = ewv[b][e, pl.ds(H + 16 * j, 16)] + g1v[b][e, pl.ds(16 * j, 16)]
          gate = 1.0 / (1.0 + jnp.exp(gn))
          m_v[e, pl.ds(16 * j, 16)] = h * f * gate

      pltpu.sync_copy(m_v, agg_sh.at[rowc[b]], add=True)
    return carry

  lax.fori_loop(0, NCHUNK // 2, pair, 0)
  plsc.subcore_barrier()

  for k in range(NPT // WB):
    r0 = sid * NPT + k * WB
    pltpu.sync_copy(agg_sh.at[pl.ds(r0, WB)], h0b0)
    pltpu.sync_copy(h0b0, out_hbm.at[cid, pl.ds(r0, WB)])


_edge = pl.kernel(
    _edge_body,
    out_type=jax.ShapeDtypeStruct((NC, NPAD, H), jnp.float32),
    mesh=_MESH,
    compiler_params=pltpu.CompilerParams(needs_layout_passes=False),
    scratch_types=[
        pltpu.VMEM((CB,), jnp.int32),
        pltpu.VMEM((CB,), jnp.int32),
        pltpu.VMEM((CB, 2 * H), jnp.float32),
        pltpu.VMEM((CB, H), jnp.float32),
        pltpu.VMEM((CB, H), jnp.float32),
        pltpu.VMEM((CB,), jnp.int32),
        pltpu.VMEM((CB,), jnp.int32),
        pltpu.VMEM((CB, 2 * H), jnp.float32),
        pltpu.VMEM((CB, H), jnp.float32),
        pltpu.VMEM((CB, H), jnp.float32),
        pltpu.VMEM((CB, H), jnp.float32),
        pltpu.VMEM_SHARED((NPAD, H), jnp.float32),
        pltpu.SemaphoreType.DMA,
        pltpu.SemaphoreType.DMA,
        pltpu.SemaphoreType.DMA,
        pltpu.SemaphoreType.DMA,
        pltpu.SemaphoreType.DMA,
        pltpu.SemaphoreType.DMA,
    ],
)


def _emb_body(z_ref, b_ref, emb_ref, aref_ref, h0_ref, tot_ref):
  i = pl.program_id(0)
  zb = z_ref[...]
  oh = (lax.broadcasted_iota(jnp.int32, (NB, NT), 1) == zb).astype(jnp.float32)
  h0_ref[...] = jnp.dot(oh, emb_ref[...], preferred_element_type=jnp.float32)
  er = jnp.dot(oh, aref_ref[...], preferred_element_type=jnp.float32)
  bh = (lax.broadcasted_iota(jnp.int32, (NB, G), 1) == b_ref[...]).astype(
      jnp.float32)
  part = jnp.sum(bh * er, axis=0, keepdims=True)

  @pl.when(i == 0)
  def _():
    tot_ref[...] = part

  @pl.when(i > 0)
  def _():
    tot_ref[...] += part


_emb_call = pl.pallas_call(
    _emb_body,
    grid=(N // NB,),
    in_specs=[
        pl.BlockSpec((NB, 1), lambda i: (i, 0)),
        pl.BlockSpec((NB, 1), lambda i: (i, 0)),
        pl.BlockSpec((NT, H), lambda i: (0, 0)),
        pl.BlockSpec((NT, 1), lambda i: (0, 0)),
    ],
    out_specs=[
        pl.BlockSpec((NB, H), lambda i: (i, 0)),
        pl.BlockSpec((1, G), lambda i: (0, 0)),
    ],
    out_shape=[
        jax.ShapeDtypeStruct((N, H), jnp.float32),
        jax.ShapeDtypeStruct((1, G), jnp.float32),
    ],
)


def _ew_body(d_ref, wcat_ref, ew_ref):
  i = pl.program_id(0)
  dd = d_ref[pl.ds(i * (EB // 128), EB // 128), :]
  env = 0.5 * (jnp.cos(jnp.pi * jnp.minimum(dd * (1.0 / CUT), 1.0)) + 1.0)
  s = env / dd
  n3 = (lax.broadcasted_iota(jnp.int32, (NRBF, 1, 1), 0).astype(jnp.float32)
        + 1.0)
  rbf = jnp.sin(n3 * ((jnp.pi / CUT) * dd)[None]) * s[None]
  w = wcat_ref[...]
  for r in range(EB // 128):
    ew_ref[pl.ds(128 * r, 128), :] = lax.dot_general(
        rbf[:, r, :], w, (((0,), (0,)), ((), ())),
        preferred_element_type=jnp.float32)


_ew_call = pl.pallas_call(
    _ew_body,
    grid=(E // EB,),
    in_specs=[
        pl.BlockSpec((E // 128, 128), lambda i: (0, 0)),
        pl.BlockSpec((NRBF, 2 * H), lambda i: (0, 0)),
    ],
    out_specs=pl.BlockSpec((EB, 2 * H), lambda i: (i, 0)),
    out_shape=jax.ShapeDtypeStruct((E, 2 * H), jnp.float32),
)


def _g1_body(h0_ref, w_ref, o_ref):
  o_ref[...] = -jnp.dot(h0_ref[...], w_ref[...],
                        preferred_element_type=jnp.float32)


_g1_call = pl.pallas_call(
    _g1_body,
    grid=(N // NB,),
    in_specs=[
        pl.BlockSpec((NB, H), lambda i: (i, 0)),
        pl.BlockSpec((H, H), lambda i: (0, 0)),
    ],
    out_specs=pl.BlockSpec((NB, H), lambda i: (i, 0)),
    out_shape=jax.ShapeDtypeStruct((N, H), jnp.float32),
)


def _tail_body(a0_ref, a1_ref, h0_ref, wd_ref, r1_ref, b1_ref, r2_ref,
               b2_ref, b_ref, tin_ref, h0o_ref, tot_ref):
  i = pl.program_id(0)
  agg = a0_ref[0] + a1_ref[0]
  h0n = h0_ref[...] + jnp.dot(agg, wd_ref[...],
                              preferred_element_type=jnp.float32)
  h0o_ref[...] = h0n
  x = jnp.dot(h0n, r1_ref[...], preferred_element_type=jnp.float32) + b1_ref[...]
  t = x / (1.0 + jnp.exp(-x))
  ae = jnp.dot(t, r2_ref[...], preferred_element_type=jnp.float32) + b2_ref[...]
  bh = (lax.broadcasted_iota(jnp.int32, (NB, G), 1) == b_ref[...]).astype(
      jnp.float32)
  part = jnp.sum(bh * ae, axis=0, keepdims=True)

  @pl.when(i == 0)
  def _():
    tot_ref[...] = tin_ref[...] + part

  @pl.when(i > 0)
  def _():
    tot_ref[...] += part


_tail_call = pl.pallas_call(
    _tail_body,
    grid=(N // NB,),
    in_specs=[
        pl.BlockSpec((1, NB, H), lambda i: (0, i, 0)),
        pl.BlockSpec((1, NB, H), lambda i: (1, i, 0)),
        pl.BlockSpec((NB, H), lambda i: (i, 0)),
        pl.BlockSpec((H, H), lambda i: (0, 0)),
        pl.BlockSpec((H, H), lambda i: (0, 0)),
        pl.BlockSpec((1, H), lambda i: (0, 0)),
        pl.BlockSpec((H, 1), lambda i: (0, 0)),
        pl.BlockSpec((1, 1), lambda i: (0, 0)),
        pl.BlockSpec((NB, 1), lambda i: (i, 0)),
        pl.BlockSpec((1, G), lambda i: (0, 0)),
    ],
    out_specs=[
        pl.BlockSpec((NB, H), lambda i: (i, 0)),
        pl.BlockSpec((1, G), lambda i: (0, 0)),
    ],
    out_shape=[
        jax.ShapeDtypeStruct((N, H), jnp.float32),
        jax.ShapeDtypeStruct((1, G), jnp.float32),
    ],
)


def kernel(z, pos, edge_index, batch, emb, W_rbf, Wg1, Wg2, Wd, R1, b1, R2,
           b2, atomic_ref):
  row = edge_index[0]
  col = edge_index[1]
  z2 = z.reshape(N, 1)
  batch2 = batch.reshape(N, 1)

  d = _geom(pos[:, 0], pos[:, 1], pos[:, 2], row, col)
  d2 = d.reshape(E // 128, 128)
  h0, tot = _emb_call(z2, batch2, emb, atomic_ref)
  for l in range(L):
    wcat = jnp.concatenate([W_rbf[l], -Wg2[l]], axis=1)
    ew = _ew_call(d2, wcat)
    g1n = _g1_call(h0, Wg1[l])
    aggp = _edge(h0, g1n, ew, row, col)
    h0, tot = _tail_call(aggp, aggp, h0, Wd[l], R1[l], b1[l].reshape(1, H),
                         R2[l], b2[l].reshape(1, 1), batch2, tot)
  return tot.reshape(G, 1)


# speedup vs baseline: 1.1547x; 1.1547x over previous
import jax
import jax.numpy as jnp
from jax import lax
from jax.experimental import pallas as pl
from jax.experimental.pallas import tpu as pltpu
from jax.experimental.pallas import tpu_sc as plsc

N = 10000
E = 320000
H = 128
NRBF = 32
L = 2
G = 64
CUT = 5.0
NT = 11

NC = 2
NS = 16
NW = NC * NS
EPT = E // NW
CB = 40
NCHUNK = EPT // CB
NPAD = 10240
NPT = NPAD // NS
WB = 40
NB = 2000
EB = 2560

_MESH = plsc.VectorSubcoreMesh(core_axis_name="c", subcore_axis_name="s")


def _geom_body(px_hbm, py_hbm, pz_hbm, row_hbm, col_hbm, d_hbm,
               px_v, py_v, pz_v, row_v, col_v, d_v):
  cid = lax.axis_index("c")
  sid = lax.axis_index("s")
  wid = sid * NC + cid
  base = wid * EPT
  pltpu.sync_copy(px_hbm, px_v)
  pltpu.sync_copy(py_hbm, py_v)
  pltpu.sync_copy(pz_hbm, pz_v)
  pltpu.sync_copy(row_hbm.at[pl.ds(base, EPT)], row_v)
  pltpu.sync_copy(col_hbm.at[pl.ds(base, EPT)], col_v)

  def body(i, carry):
    off = i * 16
    ir = row_v[pl.ds(off, 16)]
    ic = col_v[pl.ds(off, 16)]
    dx = plsc.load_gather(px_v, [ic]) - plsc.load_gather(px_v, [ir])
    dy = plsc.load_gather(py_v, [ic]) - plsc.load_gather(py_v, [ir])
    dz = plsc.load_gather(pz_v, [ic]) - plsc.load_gather(pz_v, [ir])
    s = dx * dx + dy * dy + dz * dz
    bits = plsc.bitcast(s, jnp.int32)
    y = plsc.bitcast(0x5F3759DF - (bits >> 1), jnp.float32)
    for _ in range(3):
      y = y * (1.5 - 0.5 * s * y * y)
    d_v[pl.ds(off, 16)] = jnp.maximum(s * y, 1e-8)
    return carry

  lax.fori_loop(0, EPT // 16, body, 0)
  pltpu.sync_copy(d_v, d_hbm.at[pl.ds(base, EPT)])


_geom = pl.kernel(
    _geom_body,
    out_type=jax.ShapeDtypeStruct((E,), jnp.float32),
    mesh=_MESH,
    compiler_params=pltpu.CompilerParams(needs_layout_passes=False),
    scratch_types=[
        pltpu.VMEM((N,), jnp.float32),
        pltpu.VMEM((N,), jnp.float32),
        pltpu.VMEM((N,), jnp.float32),
        pltpu.VMEM((EPT,), jnp.int32),
        pltpu.VMEM((EPT,), jnp.int32),
        pltpu.VMEM((EPT,), jnp.float32),
    ],
)


def _edge_body(h0_hbm, g1_hbm, ew_hbm, row_hbm, col_hbm, out_hbm,
               rowc0, colc0, ew0, h0b0, g1b0,
               rowc1, colc1, ew1, h0b1, g1b1,
               m_v, agg_sh, ewsem0, gsem0, hsem0, ewsem1, gsem1, hsem1,
               idxsem0, idxsem1):
  rowc = (rowc0, rowc1)
  colc = (colc0, colc1)
  ewv = (ew0, ew1)
  h0v = (h0b0, h0b1)
  g1v = (g1b0, g1b1)
  ewsem = (ewsem0, ewsem1)
  gsem = (gsem0, gsem1)
  hsem = (hsem0, hsem1)
  idxsem = (idxsem0, idxsem1)
  cid = lax.axis_index("c")
  sid = lax.axis_index("s")
  wid = sid * NC + cid

  def zb(i, carry):
    for j in range(H // 16):
      h0b0[i, pl.ds(16 * j, 16)] = jnp.zeros((16,), jnp.float32)
    return carry

  lax.fori_loop(0, WB, zb, 0)
  for k in range(NPT // WB):
    pltpu.sync_copy(h0b0, agg_sh.at[pl.ds(sid * NPT + k * WB, WB)])
  plsc.subcore_barrier()

  def fire_idx(ci, b):
    e0 = wid * EPT + ci * CB
    pltpu.async_copy(row_hbm.at[pl.ds(e0, CB)], rowc[b], idxsem[b])
    pltpu.async_copy(col_hbm.at[pl.ds(e0, CB)], colc[b], idxsem[b])

  def wait_idx(b):
    pltpu.make_async_copy(row_hbm.at[pl.ds(0, CB)], rowc[b], idxsem[b]).wait()
    pltpu.make_async_copy(col_hbm.at[pl.ds(0, CB)], colc[b], idxsem[b]).wait()

  def fire_data(ci, b):
    e0 = wid * EPT + ci * CB
    pltpu.async_copy(ew_hbm.at[pl.ds(e0, CB)], ewv[b], ewsem[b])
    pltpu.async_copy(g1_hbm.at[rowc[b]], g1v[b], gsem[b])
    pltpu.async_copy(h0_hbm.at[colc[b]], h0v[b], hsem[b])

  fire_idx(0, 0)
  fire_idx(1, 1)
  wait_idx(0)
  fire_data(0, 0)

  def pair(k, carry):
    for b in range(2):
      ci = 2 * k + b
      nb = 1 - b

      @pl.when(ci + 1 < NCHUNK)
      def _():
        wait_idx(nb)
        fire_data(ci + 1, nb)

      pltpu.make_async_copy(ew_hbm.at[pl.ds(0, CB)], ewv[b], ewsem[b]).wait()
      pltpu.make_async_copy(g1_hbm.at[rowc[b]], g1v[b], gsem[b]).wait()
      pltpu.make_async_copy(h0_hbm.at[colc[b]], h0v[b], hsem[b]).wait()

      @plsc.parallel_loop(0, CB, 1, unroll=2)
      def _(e):
        for j in range(H // 16):
          h = h0v[b][e, pl.ds(16 * j, 16)]
          f = ewv[b][e, pl.ds(16 * j, 16)]
          gn = ewv[b][e, pl.ds(H + 16 * j, 16)] + g1v[b][e, pl.ds(16 * j, 16)]
          gate = 1.0 / (1.0 + jnp.exp(gn))
          m_v[e, pl.ds(16 * j, 16)] = h * f * gate

      pltpu.sync_copy(m_v, agg_sh.at[rowc[b]], add=True)

      @pl.when(ci + 2 < NCHUNK)
      def _():
        fire_idx(ci + 2, b)
    return carry

  lax.fori_loop(0, NCHUNK // 2, pair, 0)
  plsc.subcore_barrier()

  for k in range(NPT // WB):
    r0 = sid * NPT + k * WB
    pltpu.sync_copy(agg_sh.at[pl.ds(r0, WB)], h0b0)
    pltpu.sync_copy(h0b0, out_hbm.at[cid, pl.ds(r0, WB)])


_edge = pl.kernel(
    _edge_body,
    out_type=jax.ShapeDtypeStruct((NC, NPAD, H), jnp.float32),
    mesh=_MESH,
    compiler_params=pltpu.CompilerParams(needs_layout_passes=False),
    scratch_types=[
        pltpu.VMEM((CB,), jnp.int32),
        pltpu.VMEM((CB,), jnp.int32),
        pltpu.VMEM((CB, 2 * H), jnp.float32),
        pltpu.VMEM((CB, H), jnp.float32),
        pltpu.VMEM((CB, H), jnp.float32),
        pltpu.VMEM((CB,), jnp.int32),
        pltpu.VMEM((CB,), jnp.int32),
        pltpu.VMEM((CB, 2 * H), jnp.float32),
        pltpu.VMEM((CB, H), jnp.float32),
        pltpu.VMEM((CB, H), jnp.float32),
        pltpu.VMEM((CB, H), jnp.float32),
        pltpu.VMEM_SHARED((NPAD, H), jnp.float32),
        pltpu.SemaphoreType.DMA,
        pltpu.SemaphoreType.DMA,
        pltpu.SemaphoreType.DMA,
        pltpu.SemaphoreType.DMA,
        pltpu.SemaphoreType.DMA,
        pltpu.SemaphoreType.DMA,
        pltpu.SemaphoreType.DMA,
        pltpu.SemaphoreType.DMA,
    ],
)


def _emb_body(z_ref, b_ref, emb_ref, aref_ref, h0_ref, tot_ref):
  i = pl.program_id(0)
  zb = z_ref[...]
  oh = (lax.broadcasted_iota(jnp.int32, (NB, NT), 1) == zb).astype(jnp.float32)
  h0_ref[...] = jnp.dot(oh, emb_ref[...], preferred_element_type=jnp.float32)
  er = jnp.dot(oh, aref_ref[...], preferred_element_type=jnp.float32)
  bh = (lax.broadcasted_iota(jnp.int32, (NB, G), 1) == b_ref[...]).astype(
      jnp.float32)
  part = jnp.sum(bh * er, axis=0, keepdims=True)

  @pl.when(i == 0)
  def _():
    tot_ref[...] = part

  @pl.when(i > 0)
  def _():
    tot_ref[...] += part


_emb_call = pl.pallas_call(
    _emb_body,
    grid=(N // NB,),
    in_specs=[
        pl.BlockSpec((NB, 1), lambda i: (i, 0)),
        pl.BlockSpec((NB, 1), lambda i: (i, 0)),
        pl.BlockSpec((NT, H), lambda i: (0, 0)),
        pl.BlockSpec((NT, 1), lambda i: (0, 0)),
    ],
    out_specs=[
        pl.BlockSpec((NB, H), lambda i: (i, 0)),
        pl.BlockSpec((1, G), lambda i: (0, 0)),
    ],
    out_shape=[
        jax.ShapeDtypeStruct((N, H), jnp.float32),
        jax.ShapeDtypeStruct((1, G), jnp.float32),
    ],
)


def _ew_body(d_ref, wcat_ref, ew_ref):
  i = pl.program_id(0)
  dd = d_ref[pl.ds(i * (EB // 128), EB // 128), :]
  env = 0.5 * (jnp.cos(jnp.pi * jnp.minimum(dd * (1.0 / CUT), 1.0)) + 1.0)
  s = env / dd
  n3 = (lax.broadcasted_iota(jnp.int32, (NRBF, 1, 1), 0).astype(jnp.float32)
        + 1.0)
  rbf = jnp.sin(n3 * ((jnp.pi / CUT) * dd)[None]) * s[None]
  w = wcat_ref[...]
  for r in range(EB // 128):
    ew_ref[pl.ds(128 * r, 128), :] = lax.dot_general(
        rbf[:, r, :], w, (((0,), (0,)), ((), ())),
        preferred_element_type=jnp.float32)


_ew_call = pl.pallas_call(
    _ew_body,
    grid=(E // EB,),
    in_specs=[
        pl.BlockSpec((E // 128, 128), lambda i: (0, 0)),
        pl.BlockSpec((NRBF, 2 * H), lambda i: (0, 0)),
    ],
    out_specs=pl.BlockSpec((EB, 2 * H), lambda i: (i, 0)),
    out_shape=jax.ShapeDtypeStruct((E, 2 * H), jnp.float32),
)


def _g1_body(h0_ref, w_ref, o_ref):
  o_ref[...] = -jnp.dot(h0_ref[...], w_ref[...],
                        preferred_element_type=jnp.float32)


_g1_call = pl.pallas_call(
    _g1_body,
    grid=(N // NB,),
    in_specs=[
        pl.BlockSpec((NB, H), lambda i: (i, 0)),
        pl.BlockSpec((H, H), lambda i: (0, 0)),
    ],
    out_specs=pl.BlockSpec((NB, H), lambda i: (i, 0)),
    out_shape=jax.ShapeDtypeStruct((N, H), jnp.float32),
)


def _tail_body(a0_ref, a1_ref, h0_ref, wd_ref, r1_ref, b1_ref, r2_ref,
               b2_ref, b_ref, tin_ref, h0o_ref, tot_ref):
  i = pl.program_id(0)
  agg = a0_ref[0] + a1_ref[0]
  h0n = h0_ref[...] + jnp.dot(agg, wd_ref[...],
                              preferred_element_type=jnp.float32)
  h0o_ref[...] = h0n
  x = jnp.dot(h0n, r1_ref[...], preferred_element_type=jnp.float32) + b1_ref[...]
  t = x / (1.0 + jnp.exp(-x))
  ae = jnp.dot(t, r2_ref[...], preferred_element_type=jnp.float32) + b2_ref[...]
  bh = (lax.broadcasted_iota(jnp.int32, (NB, G), 1) == b_ref[...]).astype(
      jnp.float32)
  part = jnp.sum(bh * ae, axis=0, keepdims=True)

  @pl.when(i == 0)
  def _():
    tot_ref[...] = tin_ref[...] + part

  @pl.when(i > 0)
  def _():
    tot_ref[...] += part


_tail_call = pl.pallas_call(
    _tail_body,
    grid=(N // NB,),
    in_specs=[
        pl.BlockSpec((1, NB, H), lambda i: (0, i, 0)),
        pl.BlockSpec((1, NB, H), lambda i: (1, i, 0)),
        pl.BlockSpec((NB, H), lambda i: (i, 0)),
        pl.BlockSpec((H, H), lambda i: (0, 0)),
        pl.BlockSpec((H, H), lambda i: (0, 0)),
        pl.BlockSpec((1, H), lambda i: (0, 0)),
        pl.BlockSpec((H, 1), lambda i: (0, 0)),
        pl.BlockSpec((1, 1), lambda i: (0, 0)),
        pl.BlockSpec((NB, 1), lambda i: (i, 0)),
        pl.BlockSpec((1, G), lambda i: (0, 0)),
    ],
    out_specs=[
        pl.BlockSpec((NB, H), lambda i: (i, 0)),
        pl.BlockSpec((1, G), lambda i: (0, 0)),
    ],
    out_shape=[
        jax.ShapeDtypeStruct((N, H), jnp.float32),
        jax.ShapeDtypeStruct((1, G), jnp.float32),
    ],
)


def kernel(z, pos, edge_index, batch, emb, W_rbf, Wg1, Wg2, Wd, R1, b1, R2,
           b2, atomic_ref):
  row = edge_index[0]
  col = edge_index[1]
  z2 = z.reshape(N, 1)
  batch2 = batch.reshape(N, 1)

  d = _geom(pos[:, 0], pos[:, 1], pos[:, 2], row, col)
  d2 = d.reshape(E // 128, 128)
  h0, tot = _emb_call(z2, batch2, emb, atomic_ref)
  for l in range(L):
    wcat = jnp.concatenate([W_rbf[l], -Wg2[l]], axis=1)
    ew = _ew_call(d2, wcat)
    g1n = _g1_call(h0, Wg1[l])
    aggp = _edge(h0, g1n, ew, row, col)
    h0, tot = _tail_call(aggp, aggp, h0, Wd[l], R1[l], b1[l].reshape(1, H),
                         R2[l], b2[l].reshape(1, 1), batch2, tot)
  return tot.reshape(G, 1)


# speedup vs baseline: 1.2334x; 1.0681x over previous
import jax
import jax.numpy as jnp
from jax import lax
from jax.experimental import pallas as pl
from jax.experimental.pallas import tpu as pltpu
from jax.experimental.pallas import tpu_sc as plsc

N = 10000
E = 320000
H = 128
NRBF = 32
L = 2
G = 64
CUT = 5.0
NT = 11

NC = 2
NS = 16
NW = NC * NS
EPT = E // NW
CB = 40
NCHUNK = EPT // CB
NPAD = 10240
NPT = NPAD // NS
WB = 40
NB = 2000
EB = 2560

_MESH = plsc.VectorSubcoreMesh(core_axis_name="c", subcore_axis_name="s")


def _geom_body(px_hbm, py_hbm, pz_hbm, row_hbm, col_hbm, d_hbm,
               px_v, py_v, pz_v, row_v, col_v, d_v):
  cid = lax.axis_index("c")
  sid = lax.axis_index("s")
  wid = sid * NC + cid
  base = wid * EPT
  pltpu.sync_copy(px_hbm, px_v)
  pltpu.sync_copy(py_hbm, py_v)
  pltpu.sync_copy(pz_hbm, pz_v)
  pltpu.sync_copy(row_hbm.at[pl.ds(base, EPT)], row_v)
  pltpu.sync_copy(col_hbm.at[pl.ds(base, EPT)], col_v)

  def body(i, carry):
    off = i * 16
    ir = row_v[pl.ds(off, 16)]
    ic = col_v[pl.ds(off, 16)]
    dx = plsc.load_gather(px_v, [ic]) - plsc.load_gather(px_v, [ir])
    dy = plsc.load_gather(py_v, [ic]) - plsc.load_gather(py_v, [ir])
    dz = plsc.load_gather(pz_v, [ic]) - plsc.load_gather(pz_v, [ir])
    s = dx * dx + dy * dy + dz * dz
    bits = plsc.bitcast(s, jnp.int32)
    y = plsc.bitcast(0x5F3759DF - (bits >> 1), jnp.float32)
    for _ in range(3):
      y = y * (1.5 - 0.5 * s * y * y)
    d_v[pl.ds(off, 16)] = jnp.maximum(s * y, 1e-8)
    return carry

  lax.fori_loop(0, EPT // 16, body, 0)
  pltpu.sync_copy(d_v, d_hbm.at[pl.ds(base, EPT)])


_geom = pl.kernel(
    _geom_body,
    out_type=jax.ShapeDtypeStruct((E,), jnp.float32),
    mesh=_MESH,
    compiler_params=pltpu.CompilerParams(needs_layout_passes=False),
    scratch_types=[
        pltpu.VMEM((N,), jnp.float32),
        pltpu.VMEM((N,), jnp.float32),
        pltpu.VMEM((N,), jnp.float32),
        pltpu.VMEM((EPT,), jnp.int32),
        pltpu.VMEM((EPT,), jnp.int32),
        pltpu.VMEM((EPT,), jnp.float32),
    ],
)


def _edge_body(h0_hbm, g1_hbm, ew_hbm, row_hbm, col_hbm, out_hbm,
               rowc0, colc0, ew0, h0b0, g1b0,
               rowc1, colc1, ew1, h0b1, g1b1,
               m_v, agg_sh, ewsem0, gsem0, hsem0, ewsem1, gsem1, hsem1,
               idxsem0, idxsem1):
  rowc = (rowc0, rowc1)
  colc = (colc0, colc1)
  ewv = (ew0, ew1)
  h0v = (h0b0, h0b1)
  g1v = (g1b0, g1b1)
  ewsem = (ewsem0, ewsem1)
  gsem = (gsem0, gsem1)
  hsem = (hsem0, hsem1)
  idxsem = (idxsem0, idxsem1)
  cid = lax.axis_index("c")
  sid = lax.axis_index("s")
  wid = sid * NC + cid

  def zb(i, carry):
    for j in range(H // 16):
      h0b0[i, pl.ds(16 * j, 16)] = jnp.zeros((16,), jnp.float32)
    return carry

  lax.fori_loop(0, WB, zb, 0)
  for k in range(NPT // WB):
    pltpu.sync_copy(h0b0, agg_sh.at[pl.ds(sid * NPT + k * WB, WB)])
  plsc.subcore_barrier()

  def fire_idx(ci, b):
    e0 = wid * EPT + ci * CB
    pltpu.async_copy(row_hbm.at[pl.ds(e0, CB)], rowc[b], idxsem[b])
    pltpu.async_copy(col_hbm.at[pl.ds(e0, CB)], colc[b], idxsem[b])

  def wait_idx(b):
    pltpu.make_async_copy(row_hbm.at[pl.ds(0, CB)], rowc[b], idxsem[b]).wait()
    pltpu.make_async_copy(col_hbm.at[pl.ds(0, CB)], colc[b], idxsem[b]).wait()

  def fire_data(ci, b):
    e0 = wid * EPT + ci * CB
    pltpu.async_copy(ew_hbm.at[pl.ds(e0, CB)], ewv[b], ewsem[b])
    pltpu.async_copy(g1_hbm.at[rowc[b]], g1v[b], gsem[b])
    pltpu.async_copy(h0_hbm.at[colc[b]], h0v[b], hsem[b])

  fire_idx(0, 0)
  fire_idx(1, 1)
  wait_idx(0)
  fire_data(0, 0)

  def pair(k, carry):
    for b in range(2):
      ci = 2 * k + b
      nb = 1 - b

      @pl.when(ci + 1 < NCHUNK)
      def _():
        wait_idx(nb)
        fire_data(ci + 1, nb)

      pltpu.make_async_copy(ew_hbm.at[pl.ds(0, CB)], ewv[b], ewsem[b]).wait()
      pltpu.make_async_copy(g1_hbm.at[rowc[b]], g1v[b], gsem[b]).wait()
      pltpu.make_async_copy(h0_hbm.at[colc[b]], h0v[b], hsem[b]).wait()

      @plsc.parallel_loop(0, CB, 1, unroll=2)
      def _(e):
        for j in range(H // 16):
          h = h0v[b][e, pl.ds(16 * j, 16)]
          u = ewv[b][e, pl.ds(16 * j, 16)]
          f = plsc.bitcast(u << 16, jnp.float32)
          g2n = plsc.bitcast(u & -65536, jnp.float32)
          gn = g2n + g1v[b][e, pl.ds(16 * j, 16)]
          gate = 1.0 / (1.0 + jnp.exp(gn))
          m_v[e, pl.ds(16 * j, 16)] = h * f * gate

      pltpu.sync_copy(m_v, agg_sh.at[rowc[b]], add=True)

      @pl.when(ci + 2 < NCHUNK)
      def _():
        fire_idx(ci + 2, b)
    return carry

  lax.fori_loop(0, NCHUNK // 2, pair, 0)
  plsc.subcore_barrier()

  for k in range(NPT // WB):
    r0 = sid * NPT + k * WB
    pltpu.sync_copy(agg_sh.at[pl.ds(r0, WB)], h0b0)
    pltpu.sync_copy(h0b0, out_hbm.at[cid, pl.ds(r0, WB)])


_edge = pl.kernel(
    _edge_body,
    out_type=jax.ShapeDtypeStruct((NC, NPAD, H), jnp.float32),
    mesh=_MESH,
    compiler_params=pltpu.CompilerParams(needs_layout_passes=False),
    scratch_types=[
        pltpu.VMEM((CB,), jnp.int32),
        pltpu.VMEM((CB,), jnp.int32),
        pltpu.VMEM((CB, H), jnp.int32),
        pltpu.VMEM((CB, H), jnp.float32),
        pltpu.VMEM((CB, H), jnp.float32),
        pltpu.VMEM((CB,), jnp.int32),
        pltpu.VMEM((CB,), jnp.int32),
        pltpu.VMEM((CB, H), jnp.int32),
        pltpu.VMEM((CB, H), jnp.float32),
        pltpu.VMEM((CB, H), jnp.float32),
        pltpu.VMEM((CB, H), jnp.float32),
        pltpu.VMEM_SHARED((NPAD, H), jnp.float32),
        pltpu.SemaphoreType.DMA,
        pltpu.SemaphoreType.DMA,
        pltpu.SemaphoreType.DMA,
        pltpu.SemaphoreType.DMA,
        pltpu.SemaphoreType.DMA,
        pltpu.SemaphoreType.DMA,
        pltpu.SemaphoreType.DMA,
        pltpu.SemaphoreType.DMA,
    ],
)


def _emb_body(z_ref, b_ref, emb_ref, aref_ref, h0_ref, tot_ref):
  i = pl.program_id(0)
  zb = z_ref[...]
  oh = (lax.broadcasted_iota(jnp.int32, (NB, NT), 1) == zb).astype(jnp.float32)
  h0_ref[...] = jnp.dot(oh, emb_ref[...], preferred_element_type=jnp.float32)
  er = jnp.dot(oh, aref_ref[...], preferred_element_type=jnp.float32)
  bh = (lax.broadcasted_iota(jnp.int32, (NB, G), 1) == b_ref[...]).astype(
      jnp.float32)
  part = jnp.sum(bh * er, axis=0, keepdims=True)

  @pl.when(i == 0)
  def _():
    tot_ref[...] = part

  @pl.when(i > 0)
  def _():
    tot_ref[...] += part


_emb_call = pl.pallas_call(
    _emb_body,
    grid=(N // NB,),
    in_specs=[
        pl.BlockSpec((NB, 1), lambda i: (i, 0)),
        pl.BlockSpec((NB, 1), lambda i: (i, 0)),
        pl.BlockSpec((NT, H), lambda i: (0, 0)),
        pl.BlockSpec((NT, 1), lambda i: (0, 0)),
    ],
    out_specs=[
        pl.BlockSpec((NB, H), lambda i: (i, 0)),
        pl.BlockSpec((1, G), lambda i: (0, 0)),
    ],
    out_shape=[
        jax.ShapeDtypeStruct((N, H), jnp.float32),
        jax.ShapeDtypeStruct((1, G), jnp.float32),
    ],
)


def _ew_body(d_ref, wcat_ref, ew_ref):
  i = pl.program_id(0)
  dd = d_ref[pl.ds(i * (EB // 128), EB // 128), :]
  env = 0.5 * (jnp.cos(jnp.pi * jnp.minimum(dd * (1.0 / CUT), 1.0)) + 1.0)
  s = env / dd
  n3 = (lax.broadcasted_iota(jnp.int32, (NRBF, 1, 1), 0).astype(jnp.float32)
        + 1.0)
  rbf = jnp.sin(n3 * ((jnp.pi / CUT) * dd)[None]) * s[None]
  w = wcat_ref[...]
  for r in range(EB // 128):
    x = lax.dot_general(rbf[:, r, :], w, (((0,), (0,)), ((), ())),
                        preferred_element_type=jnp.float32)
    lou = lax.bitcast_convert_type(
        x[:, :H].astype(jnp.bfloat16), jnp.uint16).astype(jnp.uint32)
    hiu = lax.bitcast_convert_type(
        x[:, H:].astype(jnp.bfloat16), jnp.uint16).astype(jnp.uint32)
    ew_ref[pl.ds(128 * r, 128), :] = lax.bitcast_convert_type(
        lou | (hiu << 16), jnp.int32)


_ew_call = pl.pallas_call(
    _ew_body,
    grid=(E // EB,),
    in_specs=[
        pl.BlockSpec((E // 128, 128), lambda i: (0, 0)),
        pl.BlockSpec((NRBF, 2 * H), lambda i: (0, 0)),
    ],
    out_specs=pl.BlockSpec((EB, H), lambda i: (i, 0)),
    out_shape=jax.ShapeDtypeStruct((E, H), jnp.int32),
)


def _g1_body(h0_ref, w_ref, o_ref):
  o_ref[...] = -jnp.dot(h0_ref[...], w_ref[...],
                        preferred_element_type=jnp.float32)


_g1_call = pl.pallas_call(
    _g1_body,
    grid=(N // NB,),
    in_specs=[
        pl.BlockSpec((NB, H), lambda i: (i, 0)),
        pl.BlockSpec((H, H), lambda i: (0, 0)),
    ],
    out_specs=pl.BlockSpec((NB, H), lambda i: (i, 0)),
    out_shape=jax.ShapeDtypeStruct((N, H), jnp.float32),
)


def _tail_body(a0_ref, a1_ref, h0_ref, wd_ref, r1_ref, b1_ref, r2_ref,
               b2_ref, b_ref, tin_ref, h0o_ref, tot_ref):
  i = pl.program_id(0)
  agg = a0_ref[0] + a1_ref[0]
  h0n = h0_ref[...] + jnp.dot(agg, wd_ref[...],
                              preferred_element_type=jnp.float32)
  h0o_ref[...] = h0n
  x = jnp.dot(h0n, r1_ref[...], preferred_element_type=jnp.float32) + b1_ref[...]
  t = x / (1.0 + jnp.exp(-x))
  ae = jnp.dot(t, r2_ref[...], preferred_element_type=jnp.float32) + b2_ref[...]
  bh = (lax.broadcasted_iota(jnp.int32, (NB, G), 1) == b_ref[...]).astype(
      jnp.float32)
  part = jnp.sum(bh * ae, axis=0, keepdims=True)

  @pl.when(i == 0)
  def _():
    tot_ref[...] = tin_ref[...] + part

  @pl.when(i > 0)
  def _():
    tot_ref[...] += part


_tail_call = pl.pallas_call(
    _tail_body,
    grid=(N // NB,),
    in_specs=[
        pl.BlockSpec((1, NB, H), lambda i: (0, i, 0)),
        pl.BlockSpec((1, NB, H), lambda i: (1, i, 0)),
        pl.BlockSpec((NB, H), lambda i: (i, 0)),
        pl.BlockSpec((H, H), lambda i: (0, 0)),
        pl.BlockSpec((H, H), lambda i: (0, 0)),
        pl.BlockSpec((1, H), lambda i: (0, 0)),
        pl.BlockSpec((H, 1), lambda i: (0, 0)),
        pl.BlockSpec((1, 1), lambda i: (0, 0)),
        pl.BlockSpec((NB, 1), lambda i: (i, 0)),
        pl.BlockSpec((1, G), lambda i: (0, 0)),
    ],
    out_specs=[
        pl.BlockSpec((NB, H), lambda i: (i, 0)),
        pl.BlockSpec((1, G), lambda i: (0, 0)),
    ],
    out_shape=[
        jax.ShapeDtypeStruct((N, H), jnp.float32),
        jax.ShapeDtypeStruct((1, G), jnp.float32),
    ],
)


def kernel(z, pos, edge_index, batch, emb, W_rbf, Wg1, Wg2, Wd, R1, b1, R2,
           b2, atomic_ref):
  row = edge_index[0]
  col = edge_index[1]
  z2 = z.reshape(N, 1)
  batch2 = batch.reshape(N, 1)

  d = _geom(pos[:, 0], pos[:, 1], pos[:, 2], row, col)
  d2 = d.reshape(E // 128, 128)
  h0, tot = _emb_call(z2, batch2, emb, atomic_ref)
  for l in range(L):
    wcat = jnp.concatenate([W_rbf[l], -Wg2[l]], axis=1)
    ew = _ew_call(d2, wcat)
    g1n = _g1_call(h0, Wg1[l])
    aggp = _edge(h0, g1n, ew, row, col)
    h0, tot = _tail_call(aggp, aggp, h0, Wd[l], R1[l], b1[l].reshape(1, H),
                         R2[l], b2[l].reshape(1, 1), batch2, tot)
  return tot.reshape(G, 1)


# speedup vs baseline: 1.3176x; 1.0683x over previous
import jax
import jax.numpy as jnp
from jax import lax
from jax.experimental import pallas as pl
from jax.experimental.pallas import tpu as pltpu
from jax.experimental.pallas import tpu_sc as plsc

N = 10000
E = 320000
H = 128
NRBF = 32
L = 2
G = 64
CUT = 5.0
NT = 11

NC = 2
NS = 16
NW = NC * NS
EPT = E // NW
CB = 40
NCHUNK = EPT // CB
NPAD = 10240
NPT = NPAD // NS
WB = 40
NB = 2000
EB = 2560

_MESH = plsc.VectorSubcoreMesh(core_axis_name="c", subcore_axis_name="s")


def _geom_body(px_hbm, py_hbm, pz_hbm, row_hbm, col_hbm, d_hbm,
               px_v, py_v, pz_v, row_v, col_v, d_v):
  cid = lax.axis_index("c")
  sid = lax.axis_index("s")
  wid = sid * NC + cid
  base = wid * EPT
  pltpu.sync_copy(px_hbm, px_v)
  pltpu.sync_copy(py_hbm, py_v)
  pltpu.sync_copy(pz_hbm, pz_v)
  pltpu.sync_copy(row_hbm.at[pl.ds(base, EPT)], row_v)
  pltpu.sync_copy(col_hbm.at[pl.ds(base, EPT)], col_v)

  def body(i, carry):
    off = i * 16
    ir = row_v[pl.ds(off, 16)]
    ic = col_v[pl.ds(off, 16)]
    dx = plsc.load_gather(px_v, [ic]) - plsc.load_gather(px_v, [ir])
    dy = plsc.load_gather(py_v, [ic]) - plsc.load_gather(py_v, [ir])
    dz = plsc.load_gather(pz_v, [ic]) - plsc.load_gather(pz_v, [ir])
    s = dx * dx + dy * dy + dz * dz
    bits = plsc.bitcast(s, jnp.int32)
    y = plsc.bitcast(0x5F3759DF - (bits >> 1), jnp.float32)
    for _ in range(3):
      y = y * (1.5 - 0.5 * s * y * y)
    d_v[pl.ds(off, 16)] = jnp.maximum(s * y, 1e-8)
    return carry

  lax.fori_loop(0, EPT // 16, body, 0)
  pltpu.sync_copy(d_v, d_hbm.at[pl.ds(base, EPT)])


_geom = pl.kernel(
    _geom_body,
    out_type=jax.ShapeDtypeStruct((E,), jnp.float32),
    mesh=_MESH,
    compiler_params=pltpu.CompilerParams(needs_layout_passes=False),
    scratch_types=[
        pltpu.VMEM((N,), jnp.float32),
        pltpu.VMEM((N,), jnp.float32),
        pltpu.VMEM((N,), jnp.float32),
        pltpu.VMEM((EPT,), jnp.int32),
        pltpu.VMEM((EPT,), jnp.int32),
        pltpu.VMEM((EPT,), jnp.float32),
    ],
)


def _edge_body(h0_hbm, g1_hbm, ew_hbm, row_hbm, col_hbm, out_hbm,
               rowc0, colc0, ew0, h0b0, g1b0,
               rowc1, colc1, ew1, h0b1, g1b1,
               m_v, scatidx_v, agg_sh, ewsem0, gsem0, hsem0, ewsem1, gsem1,
               hsem1, idxsem0, idxsem1, scatsem):
  rowc = (rowc0, rowc1)
  colc = (colc0, colc1)
  ewv = (ew0, ew1)
  h0v = (h0b0, h0b1)
  g1v = (g1b0, g1b1)
  ewsem = (ewsem0, ewsem1)
  gsem = (gsem0, gsem1)
  hsem = (hsem0, hsem1)
  idxsem = (idxsem0, idxsem1)
  cid = lax.axis_index("c")
  sid = lax.axis_index("s")
  wid = sid * NC + cid

  def zb(i, carry):
    for j in range(H // 16):
      h0b0[i, pl.ds(16 * j, 16)] = jnp.zeros((16,), jnp.float32)
    return carry

  lax.fori_loop(0, WB, zb, 0)
  for k in range(NPT // WB):
    pltpu.sync_copy(h0b0, agg_sh.at[pl.ds(sid * NPT + k * WB, WB)])
  plsc.subcore_barrier()

  def fire_idx(ci, b):
    e0 = wid * EPT + ci * CB
    pltpu.async_copy(row_hbm.at[pl.ds(e0, CB)], rowc[b], idxsem[b])
    pltpu.async_copy(col_hbm.at[pl.ds(e0, CB)], colc[b], idxsem[b])

  def wait_idx(b):
    pltpu.make_async_copy(row_hbm.at[pl.ds(0, CB)], rowc[b], idxsem[b]).wait()
    pltpu.make_async_copy(col_hbm.at[pl.ds(0, CB)], colc[b], idxsem[b]).wait()

  def fire_data(ci, b):
    e0 = wid * EPT + ci * CB
    pltpu.async_copy(ew_hbm.at[pl.ds(e0, CB)], ewv[b], ewsem[b])
    pltpu.async_copy(g1_hbm.at[rowc[b]], g1v[b], gsem[b])
    pltpu.async_copy(h0_hbm.at[colc[b]], h0v[b], hsem[b])

  fire_idx(0, 0)
  fire_idx(1, 1)
  wait_idx(0)
  fire_data(0, 0)

  def pair(k, carry):
    for b in range(2):
      ci = 2 * k + b
      nb = 1 - b

      @pl.when(ci + 1 < NCHUNK)
      def _():
        wait_idx(nb)
        fire_data(ci + 1, nb)

      pltpu.make_async_copy(ew_hbm.at[pl.ds(0, CB)], ewv[b], ewsem[b]).wait()
      pltpu.make_async_copy(g1_hbm.at[rowc[b]], g1v[b], gsem[b]).wait()
      pltpu.make_async_copy(h0_hbm.at[colc[b]], h0v[b], hsem[b]).wait()

      @pl.when(ci >= 1)
      def _():
        pltpu.make_async_copy(m_v, agg_sh.at[scatidx_v], scatsem).wait()

      for t in (0, 16, CB - 16):
        scatidx_v[pl.ds(t, 16)] = rowc[b][pl.ds(t, 16)]

      @pl.when(ci + 2 < NCHUNK)
      def _():
        fire_idx(ci + 2, b)

      @plsc.parallel_loop(0, CB, 1, unroll=2)
      def _(e):
        for j in range(H // 16):
          h = h0v[b][e, pl.ds(16 * j, 16)]
          f = ewv[b][e, pl.ds(16 * j, 16)]
          gn = ewv[b][e, pl.ds(H + 16 * j, 16)] + g1v[b][e, pl.ds(16 * j, 16)]
          gate = 1.0 / (1.0 + jnp.exp(gn))
          m_v[e, pl.ds(16 * j, 16)] = h * f * gate

      pltpu.async_copy(m_v, agg_sh.at[scatidx_v], scatsem, add=True)
    return carry

  lax.fori_loop(0, NCHUNK // 2, pair, 0)
  pltpu.make_async_copy(m_v, agg_sh.at[scatidx_v], scatsem).wait()
  plsc.subcore_barrier()

  for k in range(NPT // WB):
    r0 = sid * NPT + k * WB
    pltpu.sync_copy(agg_sh.at[pl.ds(r0, WB)], h0b0)
    pltpu.sync_copy(h0b0, out_hbm.at[cid, pl.ds(r0, WB)])


_edge = pl.kernel(
    _edge_body,
    out_type=jax.ShapeDtypeStruct((NC, NPAD, H), jnp.float32),
    mesh=_MESH,
    compiler_params=pltpu.CompilerParams(needs_layout_passes=False),
    scratch_types=[
        pltpu.VMEM((CB,), jnp.int32),
        pltpu.VMEM((CB,), jnp.int32),
        pltpu.VMEM((CB, 2 * H), jnp.float32),
        pltpu.VMEM((CB, H), jnp.float32),
        pltpu.VMEM((CB, H), jnp.float32),
        pltpu.VMEM((CB,), jnp.int32),
        pltpu.VMEM((CB,), jnp.int32),
        pltpu.VMEM((CB, 2 * H), jnp.float32),
        pltpu.VMEM((CB, H), jnp.float32),
        pltpu.VMEM((CB, H), jnp.float32),
        pltpu.VMEM((CB, H), jnp.float32),
        pltpu.VMEM((CB,), jnp.int32),
        pltpu.VMEM_SHARED((NPAD, H), jnp.float32),
        pltpu.SemaphoreType.DMA,
        pltpu.SemaphoreType.DMA,
        pltpu.SemaphoreType.DMA,
        pltpu.SemaphoreType.DMA,
        pltpu.SemaphoreType.DMA,
        pltpu.SemaphoreType.DMA,
        pltpu.SemaphoreType.DMA,
        pltpu.SemaphoreType.DMA,
        pltpu.SemaphoreType.DMA,
    ],
)


def _emb_body(z_ref, b_ref, emb_ref, aref_ref, h0_ref, tot_ref):
  i = pl.program_id(0)
  zb = z_ref[...]
  oh = (lax.broadcasted_iota(jnp.int32, (NB, NT), 1) == zb).astype(jnp.float32)
  h0_ref[...] = jnp.dot(oh, emb_ref[...], preferred_element_type=jnp.float32)
  er = jnp.dot(oh, aref_ref[...], preferred_element_type=jnp.float32)
  bh = (lax.broadcasted_iota(jnp.int32, (NB, G), 1) == b_ref[...]).astype(
      jnp.float32)
  part = jnp.sum(bh * er, axis=0, keepdims=True)

  @pl.when(i == 0)
  def _():
    tot_ref[...] = part

  @pl.when(i > 0)
  def _():
    tot_ref[...] += part


_emb_call = pl.pallas_call(
    _emb_body,
    grid=(N // NB,),
    in_specs=[
        pl.BlockSpec((NB, 1), lambda i: (i, 0)),
        pl.BlockSpec((NB, 1), lambda i: (i, 0)),
        pl.BlockSpec((NT, H), lambda i: (0, 0)),
        pl.BlockSpec((NT, 1), lambda i: (0, 0)),
    ],
    out_specs=[
        pl.BlockSpec((NB, H), lambda i: (i, 0)),
        pl.BlockSpec((1, G), lambda i: (0, 0)),
    ],
    out_shape=[
        jax.ShapeDtypeStruct((N, H), jnp.float32),
        jax.ShapeDtypeStruct((1, G), jnp.float32),
    ],
)


def _ew_body(d_ref, wcat_ref, ew_ref):
  i = pl.program_id(0)
  dd = d_ref[pl.ds(i * (EB // 128), EB // 128), :]
  env = 0.5 * (jnp.cos(jnp.pi * jnp.minimum(dd * (1.0 / CUT), 1.0)) + 1.0)
  s = env / dd
  n3 = (lax.broadcasted_iota(jnp.int32, (NRBF, 1, 1), 0).astype(jnp.float32)
        + 1.0)
  rbf = jnp.sin(n3 * ((jnp.pi / CUT) * dd)[None]) * s[None]
  w = wcat_ref[...]
  for r in range(EB // 128):
    ew_ref[pl.ds(128 * r, 128), :] = lax.dot_general(
        rbf[:, r, :], w, (((0,), (0,)), ((), ())),
        preferred_element_type=jnp.float32)


_ew_call = pl.pallas_call(
    _ew_body,
    grid=(E // EB,),
    in_specs=[
        pl.BlockSpec((E // 128, 128), lambda i: (0, 0)),
        pl.BlockSpec((NRBF, 2 * H), lambda i: (0, 0)),
    ],
    out_specs=pl.BlockSpec((EB, 2 * H), lambda i: (i, 0)),
    out_shape=jax.ShapeDtypeStruct((E, 2 * H), jnp.float32),
)


def _g1_body(h0_ref, w_ref, o_ref):
  o_ref[...] = -jnp.dot(h0_ref[...], w_ref[...],
                        preferred_element_type=jnp.float32)


_g1_call = pl.pallas_call(
    _g1_body,
    grid=(N // NB,),
    in_specs=[
        pl.BlockSpec((NB, H), lambda i: (i, 0)),
        pl.BlockSpec((H, H), lambda i: (0, 0)),
    ],
    out_specs=pl.BlockSpec((NB, H), lambda i: (i, 0)),
    out_shape=jax.ShapeDtypeStruct((N, H), jnp.float32),
)


def _tail_body(a0_ref, a1_ref, h0_ref, wd_ref, r1_ref, b1_ref, r2_ref,
               b2_ref, b_ref, tin_ref, h0o_ref, tot_ref):
  i = pl.program_id(0)
  agg = a0_ref[0] + a1_ref[0]
  h0n = h0_ref[...] + jnp.dot(agg, wd_ref[...],
                              preferred_element_type=jnp.float32)
  h0o_ref[...] = h0n
  x = jnp.dot(h0n, r1_ref[...], preferred_element_type=jnp.float32) + b1_ref[...]
  t = x / (1.0 + jnp.exp(-x))
  ae = jnp.dot(t, r2_ref[...], preferred_element_type=jnp.float32) + b2_ref[...]
  bh = (lax.broadcasted_iota(jnp.int32, (NB, G), 1) == b_ref[...]).astype(
      jnp.float32)
  part = jnp.sum(bh * ae, axis=0, keepdims=True)

  @pl.when(i == 0)
  def _():
    tot_ref[...] = tin_ref[...] + part

  @pl.when(i > 0)
  def _():
    tot_ref[...] += part


_tail_call = pl.pallas_call(
    _tail_body,
    grid=(N // NB,),
    in_specs=[
        pl.BlockSpec((1, NB, H), lambda i: (0, i, 0)),
        pl.BlockSpec((1, NB, H), lambda i: (1, i, 0)),
        pl.BlockSpec((NB, H), lambda i: (i, 0)),
        pl.BlockSpec((H, H), lambda i: (0, 0)),
        pl.BlockSpec((H, H), lambda i: (0, 0)),
        pl.BlockSpec((1, H), lambda i: (0, 0)),
        pl.BlockSpec((H, 1), lambda i: (0, 0)),
        pl.BlockSpec((1, 1), lambda i: (0, 0)),
        pl.BlockSpec((NB, 1), lambda i: (i, 0)),
        pl.BlockSpec((1, G), lambda i: (0, 0)),
    ],
    out_specs=[
        pl.BlockSpec((NB, H), lambda i: (i, 0)),
        pl.BlockSpec((1, G), lambda i: (0, 0)),
    ],
    out_shape=[
        jax.ShapeDtypeStruct((N, H), jnp.float32),
        jax.ShapeDtypeStruct((1, G), jnp.float32),
    ],
)


def kernel(z, pos, edge_index, batch, emb, W_rbf, Wg1, Wg2, Wd, R1, b1, R2,
           b2, atomic_ref):
  row = edge_index[0]
  col = edge_index[1]
  z2 = z.reshape(N, 1)
  batch2 = batch.reshape(N, 1)

  d = _geom(pos[:, 0], pos[:, 1], pos[:, 2], row, col)
  d2 = d.reshape(E // 128, 128)
  h0, tot = _emb_call(z2, batch2, emb, atomic_ref)
  for l in range(L):
    wcat = jnp.concatenate([W_rbf[l], -Wg2[l]], axis=1)
    ew = _ew_call(d2, wcat)
    g1n = _g1_call(h0, Wg1[l])
    aggp = _edge(h0, g1n, ew, row, col)
    h0, tot = _tail_call(aggp, aggp, h0, Wd[l], R1[l], b1[l].reshape(1, H),
                         R2[l], b2[l].reshape(1, 1), batch2, tot)
  return tot.reshape(G, 1)


# speedup vs baseline: 1.4866x; 1.1283x over previous
import jax
import jax.numpy as jnp
from jax import lax
from jax.experimental import pallas as pl
from jax.experimental.pallas import tpu as pltpu
from jax.experimental.pallas import tpu_sc as plsc

N = 10000
E = 320000
H = 128
NRBF = 32
L = 2
G = 64
CUT = 5.0
NT = 11

NC = 2
NS = 16
NW = NC * NS
EPT = E // NW
CB = 40
NCHUNK = EPT // CB
NPAD = 10240
NPT = NPAD // NS
WB = 40
NB = 2000
EB = 2560

_MESH = plsc.VectorSubcoreMesh(core_axis_name="c", subcore_axis_name="s")


def _rtn_bits(x):
  bx = lax.bitcast_convert_type(x, jnp.uint32)
  return bx + 0x7FFF + ((bx >> 16) & 1)


def _pack_pair_cols(x):
  rx = _rtn_bits(x)
  u = (rx[:, :H // 2] >> 16) | (rx[:, H // 2:] & jnp.uint32(0xFFFF0000))
  return lax.bitcast_convert_type(u, jnp.int32)


def _geom_body(px_hbm, py_hbm, pz_hbm, row_hbm, col_hbm, d_hbm,
               px_v, py_v, pz_v, row_v, col_v, d_v):
  cid = lax.axis_index("c")
  sid = lax.axis_index("s")
  wid = sid * NC + cid
  base = wid * EPT
  pltpu.sync_copy(px_hbm, px_v)
  pltpu.sync_copy(py_hbm, py_v)
  pltpu.sync_copy(pz_hbm, pz_v)
  pltpu.sync_copy(row_hbm.at[pl.ds(base, EPT)], row_v)
  pltpu.sync_copy(col_hbm.at[pl.ds(base, EPT)], col_v)

  def body(i, carry):
    off = i * 16
    ir = row_v[pl.ds(off, 16)]
    ic = col_v[pl.ds(off, 16)]
    dx = plsc.load_gather(px_v, [ic]) - plsc.load_gather(px_v, [ir])
    dy = plsc.load_gather(py_v, [ic]) - plsc.load_gather(py_v, [ir])
    dz = plsc.load_gather(pz_v, [ic]) - plsc.load_gather(pz_v, [ir])
    s = dx * dx + dy * dy + dz * dz
    bits = plsc.bitcast(s, jnp.int32)
    y = plsc.bitcast(0x5F3759DF - (bits >> 1), jnp.float32)
    for _ in range(3):
      y = y * (1.5 - 0.5 * s * y * y)
    d_v[pl.ds(off, 16)] = jnp.maximum(s * y, 1e-8)
    return carry

  lax.fori_loop(0, EPT // 16, body, 0)
  pltpu.sync_copy(d_v, d_hbm.at[pl.ds(base, EPT)])


_geom = pl.kernel(
    _geom_body,
    out_type=jax.ShapeDtypeStruct((E,), jnp.float32),
    mesh=_MESH,
    compiler_params=pltpu.CompilerParams(needs_layout_passes=False),
    scratch_types=[
        pltpu.VMEM((N,), jnp.float32),
        pltpu.VMEM((N,), jnp.float32),
        pltpu.VMEM((N,), jnp.float32),
        pltpu.VMEM((EPT,), jnp.int32),
        pltpu.VMEM((EPT,), jnp.int32),
        pltpu.VMEM((EPT,), jnp.float32),
    ],
)


def _edge_body(h0_hbm, g1_hbm, ew_hbm, row_hbm, col_hbm, out_hbm,
               rowc0, colc0, ew0, h0b0, g1b0,
               rowc1, colc1, ew1, h0b1, g1b1,
               m_v, scatidx_v, agg_sh, ewsem0, gsem0, hsem0, ewsem1, gsem1,
               hsem1, idxsem0, idxsem1, scatsem):
  rowc = (rowc0, rowc1)
  colc = (colc0, colc1)
  ewv = (ew0, ew1)
  h0v = (h0b0, h0b1)
  g1v = (g1b0, g1b1)
  ewsem = (ewsem0, ewsem1)
  gsem = (gsem0, gsem1)
  hsem = (hsem0, hsem1)
  idxsem = (idxsem0, idxsem1)
  cid = lax.axis_index("c")
  sid = lax.axis_index("s")
  wid = sid * NC + cid

  def zb(i, carry):
    for j in range(H // 16):
      m_v[i, pl.ds(16 * j, 16)] = jnp.zeros((16,), jnp.float32)
    return carry

  lax.fori_loop(0, WB, zb, 0)
  for k in range(NPT // WB):
    pltpu.sync_copy(m_v, agg_sh.at[pl.ds(sid * NPT + k * WB, WB)])
  plsc.subcore_barrier()

  def fire_idx(ci, b):
    e0 = wid * EPT + ci * CB
    pltpu.async_copy(row_hbm.at[pl.ds(e0, CB)], rowc[b], idxsem[b])
    pltpu.async_copy(col_hbm.at[pl.ds(e0, CB)], colc[b], idxsem[b])

  def wait_idx(b):
    pltpu.make_async_copy(row_hbm.at[pl.ds(0, CB)], rowc[b], idxsem[b]).wait()
    pltpu.make_async_copy(col_hbm.at[pl.ds(0, CB)], colc[b], idxsem[b]).wait()

  def fire_data(ci, b):
    e0 = wid * EPT + ci * CB
    pltpu.async_copy(ew_hbm.at[pl.ds(e0, CB)], ewv[b], ewsem[b])
    pltpu.async_copy(g1_hbm.at[rowc[b]], g1v[b], gsem[b])
    pltpu.async_copy(h0_hbm.at[colc[b]], h0v[b], hsem[b])

  fire_idx(0, 0)
  fire_idx(1, 1)
  wait_idx(0)
  fire_data(0, 0)

  def pair(k, carry):
    for b in range(2):
      ci = 2 * k + b
      nb = 1 - b

      @pl.when(ci + 1 < NCHUNK)
      def _():
        wait_idx(nb)
        fire_data(ci + 1, nb)

      pltpu.make_async_copy(ew_hbm.at[pl.ds(0, CB)], ewv[b], ewsem[b]).wait()
      pltpu.make_async_copy(g1_hbm.at[rowc[b]], g1v[b], gsem[b]).wait()
      pltpu.make_async_copy(h0_hbm.at[colc[b]], h0v[b], hsem[b]).wait()

      @pl.when(ci >= 1)
      def _():
        pltpu.make_async_copy(m_v, agg_sh.at[scatidx_v], scatsem).wait()

      for t in (0, 16, CB - 16):
        scatidx_v[pl.ds(t, 16)] = rowc[b][pl.ds(t, 16)]

      @pl.when(ci + 2 < NCHUNK)
      def _():
        fire_idx(ci + 2, b)

      @plsc.parallel_loop(0, CB, 1, unroll=2)
      def _(e):
        for j in range(H // 16):
          u = ewv[b][e, pl.ds(16 * j, 16)]
          f = plsc.bitcast(u << 16, jnp.float32)
          g2n = plsc.bitcast(u & -65536, jnp.float32)
          h = h0v[b][e, pl.ds(16 * j, 16)]
          g1n = g1v[b][e, pl.ds(16 * j, 16)]
          gate = 1.0 / (1.0 + jnp.exp(g2n + g1n))
          m_v[e, pl.ds(16 * j, 16)] = h * f * gate

      pltpu.async_copy(m_v, agg_sh.at[scatidx_v], scatsem, add=True)
    return carry

  lax.fori_loop(0, NCHUNK // 2, pair, 0)
  pltpu.make_async_copy(m_v, agg_sh.at[scatidx_v], scatsem).wait()
  plsc.subcore_barrier()

  for k in range(NPT // WB):
    r0 = sid * NPT + k * WB
    pltpu.sync_copy(agg_sh.at[pl.ds(r0, WB)], m_v)
    pltpu.sync_copy(m_v, out_hbm.at[cid, pl.ds(r0, WB)])


_edge = pl.kernel(
    _edge_body,
    out_type=jax.ShapeDtypeStruct((NC, NPAD, H), jnp.float32),
    mesh=_MESH,
    compiler_params=pltpu.CompilerParams(needs_layout_passes=False),
    scratch_types=[
        pltpu.VMEM((CB,), jnp.int32),
        pltpu.VMEM((CB,), jnp.int32),
        pltpu.VMEM((CB, H), jnp.int32),
        pltpu.VMEM((CB, H), jnp.float32),
        pltpu.VMEM((CB, H), jnp.float32),
        pltpu.VMEM((CB,), jnp.int32),
        pltpu.VMEM((CB,), jnp.int32),
        pltpu.VMEM((CB, H), jnp.int32),
        pltpu.VMEM((CB, H), jnp.float32),
        pltpu.VMEM((CB, H), jnp.float32),
        pltpu.VMEM((CB, H), jnp.float32),
        pltpu.VMEM((CB,), jnp.int32),
        pltpu.VMEM_SHARED((NPAD, H), jnp.float32),
        pltpu.SemaphoreType.DMA,
        pltpu.SemaphoreType.DMA,
        pltpu.SemaphoreType.DMA,
        pltpu.SemaphoreType.DMA,
        pltpu.SemaphoreType.DMA,
        pltpu.SemaphoreType.DMA,
        pltpu.SemaphoreType.DMA,
        pltpu.SemaphoreType.DMA,
        pltpu.SemaphoreType.DMA,
    ],
)


def _emb_body(z_ref, b_ref, emb_ref, aref_ref, h0_ref, tot_ref):
  i = pl.program_id(0)
  zb = z_ref[...]
  oh = (lax.broadcasted_iota(jnp.int32, (NB, NT), 1) == zb).astype(jnp.float32)
  h0_ref[...] = jnp.dot(oh, emb_ref[...], preferred_element_type=jnp.float32)
  er = jnp.dot(oh, aref_ref[...], preferred_element_type=jnp.float32)
  bh = (lax.broadcasted_iota(jnp.int32, (NB, G), 1) == b_ref[...]).astype(
      jnp.float32)
  part = jnp.sum(bh * er, axis=0, keepdims=True)

  @pl.when(i == 0)
  def _():
    tot_ref[...] = part

  @pl.when(i > 0)
  def _():
    tot_ref[...] += part


_emb_call = pl.pallas_call(
    _emb_body,
    grid=(N // NB,),
    in_specs=[
        pl.BlockSpec((NB, 1), lambda i: (i, 0)),
        pl.BlockSpec((NB, 1), lambda i: (i, 0)),
        pl.BlockSpec((NT, H), lambda i: (0, 0)),
        pl.BlockSpec((NT, 1), lambda i: (0, 0)),
    ],
    out_specs=[
        pl.BlockSpec((NB, H), lambda i: (i, 0)),
        pl.BlockSpec((1, G), lambda i: (0, 0)),
    ],
    out_shape=[
        jax.ShapeDtypeStruct((N, H), jnp.float32),
        jax.ShapeDtypeStruct((1, G), jnp.float32),
    ],
)


def _ew_body(d_ref, wcat_ref, ew_ref):
  i = pl.program_id(0)
  dd = d_ref[pl.ds(i * (EB // 128), EB // 128), :]
  env = 0.5 * (jnp.cos(jnp.pi * jnp.minimum(dd * (1.0 / CUT), 1.0)) + 1.0)
  s = env / dd
  n3 = (lax.broadcasted_iota(jnp.int32, (NRBF, 1, 1), 0).astype(jnp.float32)
        + 1.0)
  rbf = jnp.sin(n3 * ((jnp.pi / CUT) * dd)[None]) * s[None]
  w = wcat_ref[...]
  for r in range(EB // 128):
    x = lax.dot_general(rbf[:, r, :], w, (((0,), (0,)), ((), ())),
                        preferred_element_type=jnp.float32)
    rx = _rtn_bits(x)
    ew_ref[pl.ds(128 * r, 128), :] = lax.bitcast_convert_type(
        (rx[:, :H] >> 16) | (rx[:, H:] & jnp.uint32(0xFFFF0000)), jnp.int32)


_ew_call = pl.pallas_call(
    _ew_body,
    grid=(E // EB,),
    in_specs=[
        pl.BlockSpec((E // 128, 128), lambda i: (0, 0)),
        pl.BlockSpec((NRBF, 2 * H), lambda i: (0, 0)),
    ],
    out_specs=pl.BlockSpec((EB, H), lambda i: (i, 0)),
    out_shape=jax.ShapeDtypeStruct((E, H), jnp.int32),
)


def _g1_body(h0_ref, w_ref, o_ref):
  o_ref[...] = -jnp.dot(h0_ref[...], w_ref[...],
                        preferred_element_type=jnp.float32)


_g1_call = pl.pallas_call(
    _g1_body,
    grid=(N // NB,),
    in_specs=[
        pl.BlockSpec((NB, H), lambda i: (i, 0)),
        pl.BlockSpec((H, H), lambda i: (0, 0)),
    ],
    out_specs=pl.BlockSpec((NB, H), lambda i: (i, 0)),
    out_shape=jax.ShapeDtypeStruct((N, H), jnp.float32),
)


def _tail_body(a0_ref, a1_ref, h0_ref, wd_ref, r1_ref, b1_ref, r2_ref,
               b2_ref, b_ref, tin_ref, h0o_ref, tot_ref):
  i = pl.program_id(0)
  agg = a0_ref[0] + a1_ref[0]
  h0n = h0_ref[...] + jnp.dot(agg, wd_ref[...],
                              preferred_element_type=jnp.float32)
  h0o_ref[...] = h0n
  x = jnp.dot(h0n, r1_ref[...], preferred_element_type=jnp.float32) + b1_ref[...]
  t = x / (1.0 + jnp.exp(-x))
  ae = jnp.dot(t, r2_ref[...], preferred_element_type=jnp.float32) + b2_ref[...]
  bh = (lax.broadcasted_iota(jnp.int32, (NB, G), 1) == b_ref[...]).astype(
      jnp.float32)
  part = jnp.sum(bh * ae, axis=0, keepdims=True)

  @pl.when(i == 0)
  def _():
    tot_ref[...] = tin_ref[...] + part

  @pl.when(i > 0)
  def _():
    tot_ref[...] += part


_tail_call = pl.pallas_call(
    _tail_body,
    grid=(N // NB,),
    in_specs=[
        pl.BlockSpec((1, NB, H), lambda i: (0, i, 0)),
        pl.BlockSpec((1, NB, H), lambda i: (1, i, 0)),
        pl.BlockSpec((NB, H), lambda i: (i, 0)),
        pl.BlockSpec((H, H), lambda i: (0, 0)),
        pl.BlockSpec((H, H), lambda i: (0, 0)),
        pl.BlockSpec((1, H), lambda i: (0, 0)),
        pl.BlockSpec((H, 1), lambda i: (0, 0)),
        pl.BlockSpec((1, 1), lambda i: (0, 0)),
        pl.BlockSpec((NB, 1), lambda i: (i, 0)),
        pl.BlockSpec((1, G), lambda i: (0, 0)),
    ],
    out_specs=[
        pl.BlockSpec((NB, H), lambda i: (i, 0)),
        pl.BlockSpec((1, G), lambda i: (0, 0)),
    ],
    out_shape=[
        jax.ShapeDtypeStruct((N, H), jnp.float32),
        jax.ShapeDtypeStruct((1, G), jnp.float32),
    ],
)


def kernel(z, pos, edge_index, batch, emb, W_rbf, Wg1, Wg2, Wd, R1, b1, R2,
           b2, atomic_ref):
  row = edge_index[0]
  col = edge_index[1]
  z2 = z.reshape(N, 1)
  batch2 = batch.reshape(N, 1)

  d = _geom(pos[:, 0], pos[:, 1], pos[:, 2], row, col)
  d2 = d.reshape(E // 128, 128)
  h0, tot = _emb_call(z2, batch2, emb, atomic_ref)
  for l in range(L):
    wcat = jnp.concatenate([W_rbf[l], -Wg2[l]], axis=1)
    ew = _ew_call(d2, wcat)
    g1n = _g1_call(h0, Wg1[l])
    aggp = _edge(h0, g1n, ew, row, col)
    h0, tot = _tail_call(aggp, aggp, h0, Wd[l], R1[l], b1[l].reshape(1, H),
                         R2[l], b2[l].reshape(1, 1), batch2, tot)
  return tot.reshape(G, 1)
